# baseline (speedup 1.0000x reference)
import functools

import jax
import jax.numpy as jnp
from jax import lax
from jax.experimental import pallas as pl
from jax.experimental.pallas import tpu as pltpu
from jax.experimental.pallas import tpu_sc as plsc

NC = 2
NS = 16
NW = NC * NS
CH = 128


def _make_sc_agg(n_pad, d, n_chunks, with_deg):
  rows_per_sub = n_pad // NS
  mesh = plsc.VectorSubcoreMesh(core_axis_name="c", subcore_axis_name="s")

  out_type = [jax.ShapeDtypeStruct((NC, n_pad, d), jnp.float32)]
  scratch = [
      pltpu.VMEM((n_chunks, CH), jnp.int32),
      pltpu.VMEM((n_chunks, CH), jnp.int32),
      pltpu.VMEM((CH, d), jnp.float32),
      pltpu.VMEM((CH, d), jnp.float32),
      pltpu.SemaphoreType.DMA,
      pltpu.SemaphoreType.DMA,
      pltpu.SemaphoreType.DMA,
      pltpu.SemaphoreType.DMA,
      pltpu.VMEM_SHARED((n_pad, d), jnp.float32),
  ]
  if with_deg:
    out_type.append(jax.ShapeDtypeStruct((NC, n_pad, 16), jnp.float32))
    scratch += [
        pltpu.VMEM((CH, 16), jnp.float32),
        pltpu.SemaphoreType.DMA,
        pltpu.VMEM_SHARED((n_pad, 16), jnp.float32),
    ]

  def body(*refs):
    if with_deg:
      (p_hbm, src_hbm, dst_hbm, z_hbm, z16_hbm, ones_hbm,
       agg_out, deg_out, src_v, dst_v, rows_a, rows_b,
       gsem_a, gsem_b, ssem_a, ssem_b, acc_sh, ones_v, dsem, deg_sh) = refs
    else:
      (p_hbm, src_hbm, dst_hbm, z_hbm, agg_out, src_v, dst_v, rows_a, rows_b,
       gsem_a, gsem_b, ssem_a, ssem_b, acc_sh) = refs

    c = lax.axis_index("c")
    s = lax.axis_index("s")
    w = c * NS + s
    lo = s * rows_per_sub
    rsl = pl.ds(lo, rows_per_sub)

    pltpu.sync_copy(z_hbm.at[rsl], acc_sh.at[rsl])
    if with_deg:
      pltpu.sync_copy(z16_hbm.at[rsl], deg_sh.at[rsl])
      pltpu.sync_copy(ones_hbm, ones_v)

    pltpu.sync_copy(src_hbm.at[w], src_v)
    pltpu.sync_copy(dst_hbm.at[w], dst_v)
    plsc.subcore_barrier()

    def gather(j, buf, gsem):
      pltpu.async_copy(p_hbm.at[src_v.at[j]], buf, gsem)

    def wait_gather(j, buf, gsem):
      pltpu.make_async_copy(p_hbm.at[src_v.at[j]], buf, gsem).wait()

    def scatter(j, buf, ssem):
      pltpu.async_copy(buf, acc_sh.at[dst_v.at[j]], ssem, add=True)
      if with_deg:
        pltpu.async_copy(ones_v, deg_sh.at[dst_v.at[j]], dsem, add=True)

    def wait_scatter(j, buf, ssem):
      pltpu.make_async_copy(buf, acc_sh.at[dst_v.at[j]], ssem).wait()

    gather(0, rows_a, gsem_a)
    gather(1, rows_b, gsem_b)

    @pl.loop(0, n_chunks, step=2)
    def _(j):
      wait_gather(j, rows_a, gsem_a)
      scatter(j, rows_a, ssem_a)
      wait_gather(j + 1, rows_b, gsem_b)
      scatter(j + 1, rows_b, ssem_b)

      @pl.when(j + 2 < n_chunks)
      def _():
        wait_scatter(j, rows_a, ssem_a)
        gather(j + 2, rows_a, gsem_a)
        wait_scatter(j + 1, rows_b, ssem_b)
        gather(j + 3, rows_b, gsem_b)

    wait_scatter(n_chunks - 2, rows_a, ssem_a)
    wait_scatter(n_chunks - 1, rows_b, ssem_b)
    if with_deg:
      @pl.loop(0, n_chunks)
      def _(j):
        pltpu.make_async_copy(ones_v, deg_sh.at[dst_v.at[j]], dsem).wait()

    plsc.subcore_barrier()
    pltpu.sync_copy(acc_sh.at[rsl], agg_out.at[c, rsl])
    if with_deg:
      pltpu.sync_copy(deg_sh.at[rsl], deg_out.at[c, rsl])

  return pl.kernel(
      body, out_type=out_type, mesh=mesh, scratch_types=scratch,
      compiler_params=pltpu.CompilerParams(use_tc_tiling_on_sc=False))


def _proj2_call(x, wl, wr, blk):
  n, k = x.shape
  d = wl.shape[1]

  def body(x_ref, wl_ref, wr_ref, p_ref, r_ref):
    xb = x_ref[...]
    p_ref[...] = jnp.dot(xb, wl_ref[...], preferred_element_type=jnp.float32)
    r_ref[...] = jnp.dot(xb, wr_ref[...], preferred_element_type=jnp.float32)

  return pl.pallas_call(
      body,
      grid=(n // blk,),
      in_specs=[
          pl.BlockSpec((blk, k), lambda i: (i, 0)),
          pl.BlockSpec((k, d), lambda i: (0, 0)),
          pl.BlockSpec((k, d), lambda i: (0, 0)),
      ],
      out_specs=[
          pl.BlockSpec((blk, d), lambda i: (i, 0)),
          pl.BlockSpec((blk, d), lambda i: (i, 0)),
      ],
      out_shape=[
          jax.ShapeDtypeStruct((n, d), jnp.float32),
          jax.ShapeDtypeStruct((n, d), jnp.float32),
      ],
  )(x, wl, wr)


def _mid_layer_call(agg, degp, r, b, wl, wr, blk, first):
  _, n, d = agg.shape
  do = wl.shape[1]

  def body(a_ref, g_ref, r_ref, b_ref, wl_ref, wr_ref, *o_refs):
    a = a_ref[0] + a_ref[1]
    if first:
      deg = jnp.maximum(g_ref[0] + g_ref[1], 1.0)
      dinv = 1.0 / deg
    else:
      dinv = g_ref[...]
    h = jnp.maximum(a * dinv[:, 0:1] + r_ref[...] + b_ref[...], 0.0)
    o_refs[0][...] = jnp.dot(h, wl_ref[...], preferred_element_type=jnp.float32)
    o_refs[1][...] = jnp.dot(h, wr_ref[...], preferred_element_type=jnp.float32)
    if first:
      o_refs[2][...] = dinv

  g_spec = (pl.BlockSpec((NC, blk, 16), lambda i: (0, i, 0)) if first
            else pl.BlockSpec((blk, 16), lambda i: (i, 0)))
  out_specs = [pl.BlockSpec((blk, do), lambda i: (i, 0)),
               pl.BlockSpec((blk, do), lambda i: (i, 0))]
  out_shape = [jax.ShapeDtypeStruct((n, do), jnp.float32),
               jax.ShapeDtypeStruct((n, do), jnp.float32)]
  if first:
    out_specs.append(pl.BlockSpec((blk, 16), lambda i: (i, 0)))
    out_shape.append(jax.ShapeDtypeStruct((n, 16), jnp.float32))

  return pl.pallas_call(
      body,
      grid=(n // blk,),
      in_specs=[
          pl.BlockSpec((NC, blk, d), lambda i: (0, i, 0)),
          g_spec,
          pl.BlockSpec((blk, d), lambda i: (i, 0)),
          pl.BlockSpec((1, d), lambda i: (0, 0)),
          pl.BlockSpec((d, do), lambda i: (0, 0)),
          pl.BlockSpec((d, do), lambda i: (0, 0)),
      ],
      out_specs=out_specs,
      out_shape=out_shape,
  )(agg, degp, r, b, wl, wr)


def _final_call(agg, dinv, r, b, blk):
  _, n, d = agg.shape

  def body(a_ref, g_ref, r_ref, b_ref, o_ref):
    a = a_ref[0] + a_ref[1]
    o_ref[...] = a * g_ref[...][:, 0:1] + r_ref[...] + b_ref[...]

  return pl.pallas_call(
      body,
      grid=(n // blk,),
      in_specs=[
          pl.BlockSpec((NC, blk, d), lambda i: (0, i, 0)),
          pl.BlockSpec((blk, 16), lambda i: (i, 0)),
          pl.BlockSpec((blk, d), lambda i: (i, 0)),
          pl.BlockSpec((1, d), lambda i: (0, 0)),
      ],
      out_specs=pl.BlockSpec((blk, d), lambda i: (i, 0)),
      out_shape=jax.ShapeDtypeStruct((n, d), jnp.float32),
  )(agg, dinv, r, b)


def kernel(x, edge_index, Wl1, Wr1, b1, Wl2, Wr2, b2, Wl3, Wr3, b3):
  n, d_in = x.shape
  e = edge_index.shape[1]
  d_h = Wl1.shape[1]
  n_cls = Wl3.shape[1]
  do = 16

  n_pad = -(-(n + 1) // 128) * 128
  per_tile = -(-e // (NW * 2 * CH)) * 2 * CH
  n_chunks = per_tile // CH
  e_pad = per_tile * NW

  src = edge_index[0]
  dst = edge_index[1]
  pad = e_pad - e
  pad_src = (jnp.arange(pad, dtype=jnp.int32) * 97) % n
  pad_dst = n + (jnp.arange(pad, dtype=jnp.int32) % (n_pad - n))
  srcC = jnp.concatenate([src, pad_src]).reshape(NW, n_chunks, CH)
  dstC = jnp.concatenate([dst, pad_dst]).reshape(NW, n_chunks, CH)

  zeros_d = jnp.zeros((n_pad, d_h), jnp.float32)
  zeros_16 = jnp.zeros((n_pad, 16), jnp.float32)
  ones_16 = jnp.ones((CH, 16), jnp.float32)

  wl3p = jnp.zeros((d_h, do), jnp.float32).at[:, :n_cls].set(Wl3)
  wr3p = jnp.zeros((d_h, do), jnp.float32).at[:, :n_cls].set(Wr3)
  b3p = jnp.zeros((1, do), jnp.float32).at[0, :n_cls].set(b3)

  blk = 1000

  p1, r1 = _proj2_call(x, Wl1, Wr1, blk)
  agg1, degp = _make_sc_agg(n_pad, d_h, n_chunks, True)(
      p1, srcC, dstC, zeros_d, zeros_16, ones_16)
  p2, r2, dinv = _mid_layer_call(
      agg1[:, :n], degp[:, :n], r1, b1.reshape(1, d_h), Wl2, Wr2, blk, True)

  agg2, = _make_sc_agg(n_pad, d_h, n_chunks, False)(p2, srcC, dstC, zeros_d)
  p3, r3 = _mid_layer_call(
      agg2[:, :n], dinv, r2, b2.reshape(1, d_h), wl3p, wr3p, blk, False)

  zeros_do = zeros_16 if do == 16 else jnp.zeros((n_pad, do), jnp.float32)
  agg3, = _make_sc_agg(n_pad, do, n_chunks, False)(p3, srcC, dstC, zeros_do)
  out = _final_call(agg3[:, :n], dinv, r3, b3p, blk)
  return out[:, :n_cls]


# speedup vs baseline: 1.1657x; 1.1657x over previous
import functools

import jax
import jax.numpy as jnp
from jax import lax
from jax.experimental import pallas as pl
from jax.experimental.pallas import tpu as pltpu
from jax.experimental.pallas import tpu_sc as plsc

NC = 2
NS = 16
NW = NC * NS
CH = 128
NBUF = 4


def _make_sc_agg(n_pad, d, n_chunks, with_deg):
  rows_per_sub = n_pad // NS
  mesh = plsc.VectorSubcoreMesh(core_axis_name="c", subcore_axis_name="s")

  out_type = [jax.ShapeDtypeStruct((NC, n_pad, d), jnp.float32)]
  scratch = [
      pltpu.VMEM((n_chunks, CH), jnp.int32),
      pltpu.VMEM((n_chunks, CH), jnp.int32),
  ] + [pltpu.VMEM((CH, d), jnp.float32) for _ in range(NBUF)] + [
      pltpu.SemaphoreType.DMA for _ in range(2 * NBUF)
  ] + [
      pltpu.VMEM_SHARED((n_pad, d), jnp.float32),
  ]
  if with_deg:
    out_type.append(jax.ShapeDtypeStruct((NC, n_pad, 16), jnp.float32))
    scratch += [
        pltpu.VMEM((CH, 16), jnp.float32),
        pltpu.SemaphoreType.DMA,
        pltpu.VMEM_SHARED((n_pad, 16), jnp.float32),
    ]

  def body(*refs):
    if with_deg:
      (p_hbm, src_hbm, dst_hbm, z_hbm, z16_hbm, ones_hbm,
       agg_out, deg_out, src_v, dst_v, *rest) = refs
      bufs = rest[:NBUF]
      gsems = rest[NBUF:2 * NBUF]
      ssems = rest[2 * NBUF:3 * NBUF]
      acc_sh, ones_v, dsem, deg_sh = rest[3 * NBUF:]
    else:
      (p_hbm, src_hbm, dst_hbm, z_hbm, agg_out, src_v, dst_v, *rest) = refs
      bufs = rest[:NBUF]
      gsems = rest[NBUF:2 * NBUF]
      ssems = rest[2 * NBUF:3 * NBUF]
      (acc_sh,) = rest[3 * NBUF:]

    c = lax.axis_index("c")
    s = lax.axis_index("s")
    w = c * NS + s
    lo = s * rows_per_sub
    rsl = pl.ds(lo, rows_per_sub)

    pltpu.sync_copy(z_hbm.at[rsl], acc_sh.at[rsl])
    if with_deg:
      pltpu.sync_copy(z16_hbm.at[rsl], deg_sh.at[rsl])
      pltpu.sync_copy(ones_hbm, ones_v)

    pltpu.sync_copy(src_hbm.at[w], src_v)
    pltpu.sync_copy(dst_hbm.at[w], dst_v)
    plsc.subcore_barrier()

    def gather(j, k):
      pltpu.async_copy(p_hbm.at[src_v.at[j]], bufs[k], gsems[k])

    def wait_gather(j, k):
      pltpu.make_async_copy(p_hbm.at[src_v.at[j]], bufs[k], gsems[k]).wait()

    def scatter(j, k):
      pltpu.async_copy(bufs[k], acc_sh.at[dst_v.at[j]], ssems[k], add=True)
      if with_deg:
        pltpu.async_copy(ones_v, deg_sh.at[dst_v.at[j]], dsem, add=True)

    def wait_scatter(j, k):
      pltpu.make_async_copy(bufs[k], acc_sh.at[dst_v.at[j]], ssems[k]).wait()

    for k in range(NBUF):
      gather(k, k)

    @pl.loop(0, n_chunks, step=NBUF)
    def _(j):
      for k in range(NBUF):
        wait_gather(j + k, k)
        scatter(j + k, k)
      for k in range(NBUF):
        @pl.when(j + k + NBUF < n_chunks)
        def _(k=k):
          wait_scatter(j + k, k)
          gather(j + k + NBUF, k)

    for k in range(NBUF):
      wait_scatter(n_chunks - NBUF + k, k)
    if with_deg:
      @pl.loop(0, n_chunks)
      def _(j):
        pltpu.make_async_copy(ones_v, deg_sh.at[dst_v.at[j]], dsem).wait()

    plsc.subcore_barrier()
    pltpu.sync_copy(acc_sh.at[rsl], agg_out.at[c, rsl])
    if with_deg:
      pltpu.sync_copy(deg_sh.at[rsl], deg_out.at[c, rsl])

  return pl.kernel(
      body, out_type=out_type, mesh=mesh, scratch_types=scratch,
      compiler_params=pltpu.CompilerParams(use_tc_tiling_on_sc=False))


def _proj2_call(x, wl, wr, blk):
  n, k = x.shape
  d = wl.shape[1]

  def body(x_ref, wl_ref, wr_ref, p_ref, r_ref):
    xb = x_ref[...]
    p_ref[...] = jnp.dot(xb, wl_ref[...], preferred_element_type=jnp.float32)
    r_ref[...] = jnp.dot(xb, wr_ref[...], preferred_element_type=jnp.float32)

  return pl.pallas_call(
      body,
      grid=(n // blk,),
      in_specs=[
          pl.BlockSpec((blk, k), lambda i: (i, 0)),
          pl.BlockSpec((k, d), lambda i: (0, 0)),
          pl.BlockSpec((k, d), lambda i: (0, 0)),
      ],
      out_specs=[
          pl.BlockSpec((blk, d), lambda i: (i, 0)),
          pl.BlockSpec((blk, d), lambda i: (i, 0)),
      ],
      out_shape=[
          jax.ShapeDtypeStruct((n, d), jnp.float32),
          jax.ShapeDtypeStruct((n, d), jnp.float32),
      ],
  )(x, wl, wr)


def _mid_layer_call(agg, degp, r, b, wl, wr, blk, first):
  _, n, d = agg.shape
  do = wl.shape[1]

  def body(a_ref, g_ref, r_ref, b_ref, wl_ref, wr_ref, *o_refs):
    a = a_ref[0] + a_ref[1]
    if first:
      deg = jnp.maximum(g_ref[0] + g_ref[1], 1.0)
      dinv = 1.0 / deg
    else:
      dinv = g_ref[...]
    h = jnp.maximum(a * dinv[:, 0:1] + r_ref[...] + b_ref[...], 0.0)
    o_refs[0][...] = jnp.dot(h, wl_ref[...], preferred_element_type=jnp.float32)
    o_refs[1][...] = jnp.dot(h, wr_ref[...], preferred_element_type=jnp.float32)
    if first:
      o_refs[2][...] = dinv

  g_spec = (pl.BlockSpec((NC, blk, 16), lambda i: (0, i, 0)) if first
            else pl.BlockSpec((blk, 16), lambda i: (i, 0)))
  out_specs = [pl.BlockSpec((blk, do), lambda i: (i, 0)),
               pl.BlockSpec((blk, do), lambda i: (i, 0))]
  out_shape = [jax.ShapeDtypeStruct((n, do), jnp.float32),
               jax.ShapeDtypeStruct((n, do), jnp.float32)]
  if first:
    out_specs.append(pl.BlockSpec((blk, 16), lambda i: (i, 0)))
    out_shape.append(jax.ShapeDtypeStruct((n, 16), jnp.float32))

  return pl.pallas_call(
      body,
      grid=(n // blk,),
      in_specs=[
          pl.BlockSpec((NC, blk, d), lambda i: (0, i, 0)),
          g_spec,
          pl.BlockSpec((blk, d), lambda i: (i, 0)),
          pl.BlockSpec((1, d), lambda i: (0, 0)),
          pl.BlockSpec((d, do), lambda i: (0, 0)),
          pl.BlockSpec((d, do), lambda i: (0, 0)),
      ],
      out_specs=out_specs,
      out_shape=out_shape,
  )(agg, degp, r, b, wl, wr)


def _final_call(agg, dinv, r, b, blk):
  _, n, d = agg.shape

  def body(a_ref, g_ref, r_ref, b_ref, o_ref):
    a = a_ref[0] + a_ref[1]
    o_ref[...] = a * g_ref[...][:, 0:1] + r_ref[...] + b_ref[...]

  return pl.pallas_call(
      body,
      grid=(n // blk,),
      in_specs=[
          pl.BlockSpec((NC, blk, d), lambda i: (0, i, 0)),
          pl.BlockSpec((blk, 16), lambda i: (i, 0)),
          pl.BlockSpec((blk, d), lambda i: (i, 0)),
          pl.BlockSpec((1, d), lambda i: (0, 0)),
      ],
      out_specs=pl.BlockSpec((blk, d), lambda i: (i, 0)),
      out_shape=jax.ShapeDtypeStruct((n, d), jnp.float32),
  )(agg, dinv, r, b)


def kernel(x, edge_index, Wl1, Wr1, b1, Wl2, Wr2, b2, Wl3, Wr3, b3):
  n, d_in = x.shape
  e = edge_index.shape[1]
  d_h = Wl1.shape[1]
  n_cls = Wl3.shape[1]
  do = 16

  n_pad = -(-(n + 1) // 128) * 128
  per_tile = -(-e // (NW * NBUF * CH)) * NBUF * CH
  n_chunks = per_tile // CH
  e_pad = per_tile * NW

  src = edge_index[0]
  dst = edge_index[1]
  pad = e_pad - e
  pad_src = (jnp.arange(pad, dtype=jnp.int32) * 97) % n
  pad_dst = n + (jnp.arange(pad, dtype=jnp.int32) % (n_pad - n))
  srcC = jnp.concatenate([src, pad_src]).reshape(NW, n_chunks, CH)
  dstC = jnp.concatenate([dst, pad_dst]).reshape(NW, n_chunks, CH)

  zeros_d = jnp.zeros((n_pad, d_h), jnp.float32)
  zeros_16 = jnp.zeros((n_pad, 16), jnp.float32)
  ones_16 = jnp.ones((CH, 16), jnp.float32)

  wl3p = jnp.zeros((d_h, do), jnp.float32).at[:, :n_cls].set(Wl3)
  wr3p = jnp.zeros((d_h, do), jnp.float32).at[:, :n_cls].set(Wr3)
  b3p = jnp.zeros((1, do), jnp.float32).at[0, :n_cls].set(b3)

  blk = 1000

  p1, r1 = _proj2_call(x, Wl1, Wr1, blk)
  agg1, degp = _make_sc_agg(n_pad, d_h, n_chunks, True)(
      p1, srcC, dstC, zeros_d, zeros_16, ones_16)
  p2, r2, dinv = _mid_layer_call(
      agg1[:, :n], degp[:, :n], r1, b1.reshape(1, d_h), Wl2, Wr2, blk, True)

  agg2, = _make_sc_agg(n_pad, d_h, n_chunks, False)(p2, srcC, dstC, zeros_d)
  p3, r3 = _mid_layer_call(
      agg2[:, :n], dinv, r2, b2.reshape(1, d_h), wl3p, wr3p, blk, False)

  zeros_do = zeros_16 if do == 16 else jnp.zeros((n_pad, do), jnp.float32)
  agg3, = _make_sc_agg(n_pad, do, n_chunks, False)(p3, srcC, dstC, zeros_do)
  out = _final_call(agg3[:, :n], dinv, r3, b3p, blk)
  return out[:, :n_cls]


# speedup vs baseline: 1.1881x; 1.0192x over previous
import functools

import jax
import jax.numpy as jnp
from jax import lax
from jax.experimental import pallas as pl
from jax.experimental.pallas import tpu as pltpu
from jax.experimental.pallas import tpu_sc as plsc

NC = 2
NS = 16
NW = NC * NS
CH = 128
NBUF = 4


def _make_sc_agg(n_pad, d, n_chunks, with_deg):
  rows_per_sub = n_pad // NS
  mesh = plsc.VectorSubcoreMesh(core_axis_name="c", subcore_axis_name="s")

  out_type = [jax.ShapeDtypeStruct((NC, n_pad, d), jnp.float32)]
  scratch = [
      pltpu.VMEM((n_chunks, CH), jnp.int32),
      pltpu.VMEM((n_chunks, CH), jnp.int32),
  ] + [pltpu.VMEM((CH, d), jnp.float32) for _ in range(NBUF)] + [
      pltpu.SemaphoreType.DMA for _ in range(2 * NBUF)
  ] + [
      pltpu.VMEM_SHARED((n_pad, d), jnp.float32),
  ]
  if with_deg:
    out_type.append(jax.ShapeDtypeStruct((NC, n_pad, 16), jnp.float32))
    scratch += [
        pltpu.VMEM((CH, 16), jnp.float32),
        pltpu.SemaphoreType.DMA,
        pltpu.VMEM_SHARED((n_pad, 16), jnp.float32),
    ]

  def body(*refs):
    if with_deg:
      (p_hbm, src_hbm, dst_hbm, z_hbm, z16_hbm, ones_hbm,
       agg_out, deg_out, src_v, dst_v, *rest) = refs
      bufs = rest[:NBUF]
      gsems = rest[NBUF:2 * NBUF]
      ssems = rest[2 * NBUF:3 * NBUF]
      acc_sh, ones_v, dsem, deg_sh = rest[3 * NBUF:]
    else:
      (p_hbm, src_hbm, dst_hbm, z_hbm, agg_out, src_v, dst_v, *rest) = refs
      bufs = rest[:NBUF]
      gsems = rest[NBUF:2 * NBUF]
      ssems = rest[2 * NBUF:3 * NBUF]
      (acc_sh,) = rest[3 * NBUF:]

    c = lax.axis_index("c")
    s = lax.axis_index("s")
    w = c * NS + s
    lo = s * rows_per_sub
    rsl = pl.ds(lo, rows_per_sub)

    pltpu.async_copy(z_hbm.at[rsl], acc_sh.at[rsl], gsems[0])
    pltpu.async_copy(src_hbm.at[w], src_v, gsems[1])
    pltpu.async_copy(dst_hbm.at[w], dst_v, gsems[2])
    if with_deg:
      pltpu.async_copy(z16_hbm.at[rsl], deg_sh.at[rsl], gsems[3])
      pltpu.async_copy(ones_hbm, ones_v, ssems[0])
      pltpu.make_async_copy(z16_hbm.at[rsl], deg_sh.at[rsl], gsems[3]).wait()
      pltpu.make_async_copy(ones_hbm, ones_v, ssems[0]).wait()
    pltpu.make_async_copy(z_hbm.at[rsl], acc_sh.at[rsl], gsems[0]).wait()
    pltpu.make_async_copy(src_hbm.at[w], src_v, gsems[1]).wait()
    pltpu.make_async_copy(dst_hbm.at[w], dst_v, gsems[2]).wait()
    plsc.subcore_barrier()

    def gather(j, k):
      pltpu.async_copy(p_hbm.at[src_v.at[j]], bufs[k], gsems[k])

    def wait_gather(j, k):
      pltpu.make_async_copy(p_hbm.at[src_v.at[j]], bufs[k], gsems[k]).wait()

    def scatter(j, k):
      pltpu.async_copy(bufs[k], acc_sh.at[dst_v.at[j]], ssems[k], add=True)
      if with_deg:
        pltpu.async_copy(ones_v, deg_sh.at[dst_v.at[j]], dsem, add=True)

    def wait_scatter(j, k):
      pltpu.make_async_copy(bufs[k], acc_sh.at[dst_v.at[j]], ssems[k]).wait()

    for k in range(NBUF):
      gather(k, k)

    @pl.loop(0, n_chunks, step=NBUF)
    def _(j):
      for k in range(NBUF):
        wait_gather(j + k, k)
        scatter(j + k, k)
      for k in range(NBUF):
        @pl.when(j + k + NBUF < n_chunks)
        def _(k=k):
          wait_scatter(j + k, k)
          gather(j + k + NBUF, k)

    for k in range(NBUF):
      wait_scatter(n_chunks - NBUF + k, k)
    if with_deg:
      @pl.loop(0, n_chunks)
      def _(j):
        pltpu.make_async_copy(ones_v, deg_sh.at[dst_v.at[j]], dsem).wait()

    plsc.subcore_barrier()
    pltpu.sync_copy(acc_sh.at[rsl], agg_out.at[c, rsl])
    if with_deg:
      pltpu.sync_copy(deg_sh.at[rsl], deg_out.at[c, rsl])

  return pl.kernel(
      body, out_type=out_type, mesh=mesh, scratch_types=scratch,
      compiler_params=pltpu.CompilerParams(use_tc_tiling_on_sc=False))


def _proj_call(x, wl, blk):
  n, k = x.shape
  d = wl.shape[1]

  def body(x_ref, wl_ref, p_ref):
    p_ref[...] = jnp.dot(x_ref[...], wl_ref[...],
                         preferred_element_type=jnp.float32)

  return pl.pallas_call(
      body,
      grid=(n // blk,),
      in_specs=[
          pl.BlockSpec((blk, k), lambda i: (i, 0)),
          pl.BlockSpec((k, d), lambda i: (0, 0)),
      ],
      out_specs=pl.BlockSpec((blk, d), lambda i: (i, 0)),
      out_shape=jax.ShapeDtypeStruct((n, d), jnp.float32),
  )(x, wl)


def _mid_layer_call(agg, degp, r, b, wl, blk, first):
  _, n, d = agg.shape
  do = wl.shape[1]

  def body(a_ref, g_ref, r_ref, b_ref, wl_ref, *o_refs):
    a = a_ref[0] + a_ref[1]
    if first:
      deg = jnp.maximum(g_ref[0] + g_ref[1], 1.0)
      dinv = 1.0 / deg
    else:
      dinv = g_ref[...]
    h = jnp.maximum(a * dinv[:, 0:1] + r_ref[...] + b_ref[...], 0.0)
    o_refs[0][...] = h
    o_refs[1][...] = jnp.dot(h, wl_ref[...], preferred_element_type=jnp.float32)
    if first:
      o_refs[2][...] = dinv

  g_spec = (pl.BlockSpec((NC, blk, 16), lambda i: (0, i, 0)) if first
            else pl.BlockSpec((blk, 16), lambda i: (i, 0)))
  out_specs = [pl.BlockSpec((blk, d), lambda i: (i, 0)),
               pl.BlockSpec((blk, do), lambda i: (i, 0))]
  out_shape = [jax.ShapeDtypeStruct((n, d), jnp.float32),
               jax.ShapeDtypeStruct((n, do), jnp.float32)]
  if first:
    out_specs.append(pl.BlockSpec((blk, 16), lambda i: (i, 0)))
    out_shape.append(jax.ShapeDtypeStruct((n, 16), jnp.float32))

  return pl.pallas_call(
      body,
      grid=(n // blk,),
      in_specs=[
          pl.BlockSpec((NC, blk, d), lambda i: (0, i, 0)),
          g_spec,
          pl.BlockSpec((blk, d), lambda i: (i, 0)),
          pl.BlockSpec((1, d), lambda i: (0, 0)),
          pl.BlockSpec((d, do), lambda i: (0, 0)),
      ],
      out_specs=out_specs,
      out_shape=out_shape,
  )(agg, degp, r, b, wl)


def _final_call(agg, dinv, r, b, blk):
  _, n, d = agg.shape

  def body(a_ref, g_ref, r_ref, b_ref, o_ref):
    a = a_ref[0] + a_ref[1]
    o_ref[...] = a * g_ref[...][:, 0:1] + r_ref[...] + b_ref[...]

  return pl.pallas_call(
      body,
      grid=(n // blk,),
      in_specs=[
          pl.BlockSpec((NC, blk, d), lambda i: (0, i, 0)),
          pl.BlockSpec((blk, 16), lambda i: (i, 0)),
          pl.BlockSpec((blk, d), lambda i: (i, 0)),
          pl.BlockSpec((1, d), lambda i: (0, 0)),
      ],
      out_specs=pl.BlockSpec((blk, d), lambda i: (i, 0)),
      out_shape=jax.ShapeDtypeStruct((n, d), jnp.float32),
  )(agg, dinv, r, b)


def kernel(x, edge_index, Wl1, Wr1, b1, Wl2, Wr2, b2, Wl3, Wr3, b3):
  n, d_in = x.shape
  e = edge_index.shape[1]
  d_h = Wl1.shape[1]
  n_cls = Wl3.shape[1]
  do = 16

  n_pad = -(-(n + 1) // 128) * 128
  per_tile = -(-e // (NW * NBUF * CH)) * NBUF * CH
  n_chunks = per_tile // CH
  e_pad = per_tile * NW

  src = edge_index[0]
  dst = edge_index[1]
  pad = e_pad - e
  pad_src = (jnp.arange(pad, dtype=jnp.int32) * 97) % n
  pad_dst = n + (jnp.arange(pad, dtype=jnp.int32) % (n_pad - n))
  srcC = jnp.concatenate([src, pad_src]).reshape(NW, n_chunks, CH)
  dstC = jnp.concatenate([dst, pad_dst]).reshape(NW, n_chunks, CH)

  zeros_d = jnp.zeros((n_pad, d_h), jnp.float32)
  zeros_16 = jnp.zeros((n_pad, 16), jnp.float32)
  ones_16 = jnp.ones((CH, 16), jnp.float32)

  wl3p = jnp.zeros((d_h, do), jnp.float32).at[:, :n_cls].set(Wl3)
  wr3p = jnp.zeros((d_h, do), jnp.float32).at[:, :n_cls].set(Wr3)
  b3p = jnp.zeros((1, do), jnp.float32).at[0, :n_cls].set(b3)

  blk = 1000

  p1 = _proj_call(x, Wl1, blk)
  agg1, degp = _make_sc_agg(n_pad, d_h, n_chunks, True)(
      p1, srcC, dstC, zeros_d, zeros_16, ones_16)
  r1 = _proj_call(x, Wr1, blk)
  h1, p2, dinv = _mid_layer_call(
      agg1[:, :n], degp[:, :n], r1, b1.reshape(1, d_h), Wl2, blk, True)

  agg2, = _make_sc_agg(n_pad, d_h, n_chunks, False)(p2, srcC, dstC, zeros_d)
  r2 = _proj_call(h1, Wr2, blk)
  h2, p3 = _mid_layer_call(
      agg2[:, :n], dinv, r2, b2.reshape(1, d_h), wl3p, blk, False)

  zeros_do = zeros_16 if do == 16 else jnp.zeros((n_pad, do), jnp.float32)
  agg3, = _make_sc_agg(n_pad, do, n_chunks, False)(p3, srcC, dstC, zeros_do)
  r3 = _proj_call(h2, wr3p, blk)
  out = _final_call(agg3[:, :n], dinv, r3, b3p, blk)
  return out[:, :n_cls]


# speedup vs baseline: 1.2722x; 1.0708x over previous
import functools

import jax
import jax.numpy as jnp
from jax import lax
from jax.experimental import pallas as pl
from jax.experimental.pallas import tpu as pltpu
from jax.experimental.pallas import tpu_sc as plsc

NC = 2
NS = 16
NW = NC * NS
CH = 128
NBUF = 4


def _make_sc_agg(n_pad, d, n_chunks, ch, with_deg):
  rows_per_sub = n_pad // NS
  mesh = plsc.VectorSubcoreMesh(core_axis_name="c", subcore_axis_name="s")

  out_type = [jax.ShapeDtypeStruct((NC, n_pad, d), jnp.float32)]
  scratch = [
      pltpu.VMEM((n_chunks, ch), jnp.int32),
      pltpu.VMEM((n_chunks, ch), jnp.int32),
  ] + [pltpu.VMEM((ch, d), jnp.float32) for _ in range(NBUF)] + [
      pltpu.SemaphoreType.DMA for _ in range(2 * NBUF)
  ] + [
      pltpu.VMEM_SHARED((n_pad, d), jnp.float32),
  ]
  if with_deg:
    out_type.append(jax.ShapeDtypeStruct((NC, n_pad, 16), jnp.float32))
    scratch += [
        pltpu.VMEM((ch, 16), jnp.float32),
        pltpu.SemaphoreType.DMA,
        pltpu.VMEM_SHARED((n_pad, 16), jnp.float32),
    ]

  def body(*refs):
    if with_deg:
      (p_hbm, src_hbm, dst_hbm, z_hbm, z16_hbm, ones_hbm,
       agg_out, deg_out, src_v, dst_v, *rest) = refs
      bufs = rest[:NBUF]
      gsems = rest[NBUF:2 * NBUF]
      ssems = rest[2 * NBUF:3 * NBUF]
      acc_sh, ones_v, dsem, deg_sh = rest[3 * NBUF:]
    else:
      (p_hbm, src_hbm, dst_hbm, z_hbm, agg_out, src_v, dst_v, *rest) = refs
      bufs = rest[:NBUF]
      gsems = rest[NBUF:2 * NBUF]
      ssems = rest[2 * NBUF:3 * NBUF]
      (acc_sh,) = rest[3 * NBUF:]

    c = lax.axis_index("c")
    s = lax.axis_index("s")
    w = c * NS + s
    lo = s * rows_per_sub
    rsl = pl.ds(lo, rows_per_sub)

    pltpu.async_copy(z_hbm.at[rsl], acc_sh.at[rsl], gsems[0])
    pltpu.async_copy(src_hbm.at[w], src_v, gsems[1])
    pltpu.async_copy(dst_hbm.at[w], dst_v, gsems[2])
    if with_deg:
      pltpu.async_copy(z16_hbm.at[rsl], deg_sh.at[rsl], gsems[3])
      pltpu.async_copy(ones_hbm, ones_v, ssems[0])
      pltpu.make_async_copy(z16_hbm.at[rsl], deg_sh.at[rsl], gsems[3]).wait()
      pltpu.make_async_copy(ones_hbm, ones_v, ssems[0]).wait()
    pltpu.make_async_copy(z_hbm.at[rsl], acc_sh.at[rsl], gsems[0]).wait()
    pltpu.make_async_copy(src_hbm.at[w], src_v, gsems[1]).wait()
    pltpu.make_async_copy(dst_hbm.at[w], dst_v, gsems[2]).wait()
    plsc.subcore_barrier()

    def gather(j, k):
      pltpu.async_copy(p_hbm.at[src_v.at[j]], bufs[k], gsems[k])

    def wait_gather(j, k):
      pltpu.make_async_copy(p_hbm.at[src_v.at[j]], bufs[k], gsems[k]).wait()

    def scatter(j, k):
      pltpu.async_copy(bufs[k], acc_sh.at[dst_v.at[j]], ssems[k], add=True)
      if with_deg:
        pltpu.async_copy(ones_v, deg_sh.at[dst_v.at[j]], dsem, add=True)

    def wait_scatter(j, k):
      pltpu.make_async_copy(bufs[k], acc_sh.at[dst_v.at[j]], ssems[k]).wait()

    for k in range(NBUF):
      gather(k, k)

    @pl.loop(0, n_chunks, step=NBUF)
    def _(j):
      for k in range(NBUF):
        wait_gather(j + k, k)
        scatter(j + k, k)
      for k in range(NBUF):
        @pl.when(j + k + NBUF < n_chunks)
        def _(k=k):
          wait_scatter(j + k, k)
          gather(j + k + NBUF, k)

    for k in range(NBUF):
      wait_scatter(n_chunks - NBUF + k, k)
    if with_deg:
      @pl.loop(0, n_chunks)
      def _(j):
        pltpu.make_async_copy(ones_v, deg_sh.at[dst_v.at[j]], dsem).wait()

    plsc.subcore_barrier()
    pltpu.sync_copy(acc_sh.at[rsl], agg_out.at[c, rsl])
    if with_deg:
      pltpu.sync_copy(deg_sh.at[rsl], deg_out.at[c, rsl])

  return pl.kernel(
      body, out_type=out_type, mesh=mesh, scratch_types=scratch,
      compiler_params=pltpu.CompilerParams(use_tc_tiling_on_sc=False))


def _proj_call(x, wl, blk):
  n, k = x.shape
  d = wl.shape[1]

  def body(x_ref, wl_ref, p_ref):
    p_ref[...] = jnp.dot(x_ref[...], wl_ref[...],
                         preferred_element_type=jnp.float32)

  return pl.pallas_call(
      body,
      grid=(n // blk,),
      in_specs=[
          pl.BlockSpec((blk, k), lambda i: (i, 0)),
          pl.BlockSpec((k, d), lambda i: (0, 0)),
      ],
      out_specs=pl.BlockSpec((blk, d), lambda i: (i, 0)),
      out_shape=jax.ShapeDtypeStruct((n, d), jnp.float32),
  )(x, wl)


def _mid_layer_call(agg, degp, r, b, wl, blk, first):
  _, n, d = agg.shape
  do = wl.shape[1]
  nblk = n // blk
  aggf = agg.reshape(NC * n, d)
  gf = degp.reshape(NC * n, 16) if first else degp

  def body(a0_ref, a1_ref, g0_ref, g1_ref, r_ref, b_ref, wl_ref, *o_refs):
    a = a0_ref[...] + a1_ref[...]
    if first:
      deg = jnp.maximum(g0_ref[...] + g1_ref[...], 1.0)
      dinv = 1.0 / deg
    else:
      dinv = g0_ref[...]
    h = jnp.maximum(a * dinv[:, 0:1] + r_ref[...] + b_ref[...], 0.0)
    o_refs[0][...] = h
    o_refs[1][...] = jnp.dot(h, wl_ref[...], preferred_element_type=jnp.float32)
    if first:
      o_refs[2][...] = dinv

  g_spec0 = pl.BlockSpec((blk, 16), lambda i: (i, 0))
  g_spec1 = (pl.BlockSpec((blk, 16), lambda i: (i + nblk, 0)) if first
             else g_spec0)
  out_specs = [pl.BlockSpec((blk, d), lambda i: (i, 0)),
               pl.BlockSpec((blk, do), lambda i: (i, 0))]
  out_shape = [jax.ShapeDtypeStruct((n, d), jnp.float32),
               jax.ShapeDtypeStruct((n, do), jnp.float32)]
  if first:
    out_specs.append(pl.BlockSpec((blk, 16), lambda i: (i, 0)))
    out_shape.append(jax.ShapeDtypeStruct((n, 16), jnp.float32))

  return pl.pallas_call(
      body,
      grid=(nblk,),
      in_specs=[
          pl.BlockSpec((blk, d), lambda i: (i, 0)),
          pl.BlockSpec((blk, d), lambda i: (i + nblk, 0)),
          g_spec0,
          g_spec1,
          pl.BlockSpec((blk, d), lambda i: (i, 0)),
          pl.BlockSpec((1, d), lambda i: (0, 0)),
          pl.BlockSpec((d, do), lambda i: (0, 0)),
      ],
      out_specs=out_specs,
      out_shape=out_shape,
  )(aggf, aggf, gf, gf, r, b, wl)


def _final_call(agg, dinv, r, b, blk):
  _, n, d = agg.shape
  nblk = n // blk
  aggf = agg.reshape(NC * n, d)

  def body(a0_ref, a1_ref, g_ref, r_ref, b_ref, o_ref):
    a = a0_ref[...] + a1_ref[...]
    o_ref[...] = a * g_ref[...][:, 0:1] + r_ref[...] + b_ref[...]

  return pl.pallas_call(
      body,
      grid=(nblk,),
      in_specs=[
          pl.BlockSpec((blk, d), lambda i: (i, 0)),
          pl.BlockSpec((blk, d), lambda i: (i + nblk, 0)),
          pl.BlockSpec((blk, 16), lambda i: (i, 0)),
          pl.BlockSpec((blk, d), lambda i: (i, 0)),
          pl.BlockSpec((1, d), lambda i: (0, 0)),
      ],
      out_specs=pl.BlockSpec((blk, d), lambda i: (i, 0)),
      out_shape=jax.ShapeDtypeStruct((n, d), jnp.float32),
  )(aggf, aggf, dinv, r, b)


def kernel(x, edge_index, Wl1, Wr1, b1, Wl2, Wr2, b2, Wl3, Wr3, b3):
  n, d_in = x.shape
  e = edge_index.shape[1]
  d_h = Wl1.shape[1]
  n_cls = Wl3.shape[1]
  do = 16

  n_pad = -(-(n + 1) // 128) * 128
  per_tile = e // NW
  ch = CH
  while per_tile % (ch * NBUF) != 0:
    ch -= 1
  n_chunks = per_tile // ch

  srcC = edge_index[0].reshape(NW, n_chunks, ch)
  dstC = edge_index[1].reshape(NW, n_chunks, ch)

  zeros_d = jnp.zeros((n_pad, d_h), jnp.float32)
  zeros_16 = jnp.zeros((n_pad, 16), jnp.float32)
  ones_16 = jnp.ones((ch, 16), jnp.float32)

  wl3p = jnp.zeros((d_h, do), jnp.float32).at[:, :n_cls].set(Wl3)
  wr3p = jnp.zeros((d_h, do), jnp.float32).at[:, :n_cls].set(Wr3)
  b3p = jnp.zeros((1, do), jnp.float32).at[0, :n_cls].set(b3)

  blk = n_pad // 8
  xp = jnp.pad(x, ((0, n_pad - n), (0, 0)))

  p1 = _proj_call(xp, Wl1, blk)
  agg1, degp = _make_sc_agg(n_pad, d_h, n_chunks, ch, True)(
      p1, srcC, dstC, zeros_d, zeros_16, ones_16)
  r1 = _proj_call(xp, Wr1, blk)
  h1, p2, dinv = _mid_layer_call(
      agg1, degp, r1, b1.reshape(1, d_h), Wl2, blk, True)

  agg2, = _make_sc_agg(n_pad, d_h, n_chunks, ch, False)(p2, srcC, dstC, zeros_d)
  r2 = _proj_call(h1, Wr2, blk)
  h2, p3 = _mid_layer_call(
      agg2, dinv, r2, b2.reshape(1, d_h), wl3p, blk, False)

  zeros_do = zeros_16 if do == 16 else jnp.zeros((n_pad, do), jnp.float32)
  agg3, = _make_sc_agg(n_pad, do, n_chunks, ch, False)(p3, srcC, dstC, zeros_do)
  r3 = _proj_call(h2, wr3p, blk)
  out = _final_call(agg3, dinv, r3, b3p, blk)
  return out[:n, :n_cls]


# speedup vs baseline: 1.2810x; 1.0069x over previous
import functools

import jax
import jax.numpy as jnp
from jax import lax
from jax.experimental import pallas as pl
from jax.experimental.pallas import tpu as pltpu
from jax.experimental.pallas import tpu_sc as plsc

NC = 2
NS = 16
NW = NC * NS
CH = 128
NBUF = 4


def _make_sc_agg(n_pad, d, n_chunks, ch, e_tile, with_deg):
  rows_per_sub = n_pad // NS
  mesh = plsc.VectorSubcoreMesh(core_axis_name="c", subcore_axis_name="s")

  out_type = [jax.ShapeDtypeStruct((NC, n_pad, d), jnp.float32)]
  slab = n_chunks * ch
  scratch = [
      pltpu.VMEM((slab,), jnp.int32),
      pltpu.VMEM((slab,), jnp.int32),
  ] + [pltpu.VMEM((ch, d), jnp.float32) for _ in range(NBUF)] + [
      pltpu.SemaphoreType.DMA for _ in range(2 * NBUF)
  ] + [
      pltpu.VMEM_SHARED((n_pad, d), jnp.float32),
  ]
  if with_deg:
    out_type.append(jax.ShapeDtypeStruct((NC, n_pad, 16), jnp.float32))
    scratch += [
        pltpu.VMEM((ch, 16), jnp.float32),
        pltpu.SemaphoreType.DMA,
        pltpu.VMEM_SHARED((n_pad, 16), jnp.float32),
    ]

  def body(*refs):
    if with_deg:
      (p_hbm, src_hbm, dst_hbm, z_hbm, z16_hbm, ones_hbm,
       agg_out, deg_out, src_v, dst_v, *rest) = refs
      bufs = rest[:NBUF]
      gsems = rest[NBUF:2 * NBUF]
      ssems = rest[2 * NBUF:3 * NBUF]
      acc_sh, ones_v, dsem, deg_sh = rest[3 * NBUF:]
    else:
      (p_hbm, src_hbm, dst_hbm, z_hbm, agg_out, src_v, dst_v, *rest) = refs
      bufs = rest[:NBUF]
      gsems = rest[NBUF:2 * NBUF]
      ssems = rest[2 * NBUF:3 * NBUF]
      (acc_sh,) = rest[3 * NBUF:]

    c = lax.axis_index("c")
    s = lax.axis_index("s")
    w = c * NS + s
    lo = s * rows_per_sub
    rsl = pl.ds(lo, rows_per_sub)
    esl = pl.ds(w * e_tile, e_tile)
    vsl = pl.ds(0, e_tile)

    pltpu.async_copy(z_hbm.at[rsl], acc_sh.at[rsl], gsems[0])
    pltpu.async_copy(src_hbm.at[esl], src_v.at[vsl], gsems[1])
    pltpu.async_copy(dst_hbm.at[esl], dst_v.at[vsl], gsems[2])
    if with_deg:
      pltpu.async_copy(z16_hbm.at[rsl], deg_sh.at[rsl], gsems[3])
      pltpu.async_copy(ones_hbm, ones_v, ssems[0])
      pltpu.make_async_copy(z16_hbm.at[rsl], deg_sh.at[rsl], gsems[3]).wait()
      pltpu.make_async_copy(ones_hbm, ones_v, ssems[0]).wait()
    pltpu.make_async_copy(z_hbm.at[rsl], acc_sh.at[rsl], gsems[0]).wait()
    pltpu.make_async_copy(src_hbm.at[esl], src_v.at[vsl], gsems[1]).wait()
    pltpu.make_async_copy(dst_hbm.at[esl], dst_v.at[vsl], gsems[2]).wait()
    iota = lax.iota(jnp.int32, 16)
    for t in range((slab - e_tile) // 16):
      tsl = pl.ds(e_tile + t * 16, 16)
      src_v[tsl] = iota + s * 16
      dst_v[tsl] = iota + (n_pad - 16)
    plsc.subcore_barrier()

    def gather(j, k):
      pltpu.async_copy(p_hbm.at[src_v.at[pl.ds(j * ch, ch)]], bufs[k], gsems[k])

    def wait_gather(j, k):
      pltpu.make_async_copy(
          p_hbm.at[src_v.at[pl.ds(j * ch, ch)]], bufs[k], gsems[k]).wait()

    def scatter(j, k):
      dsl = dst_v.at[pl.ds(j * ch, ch)]
      pltpu.async_copy(bufs[k], acc_sh.at[dsl], ssems[k], add=True)
      if with_deg:
        pltpu.async_copy(ones_v, deg_sh.at[dsl], dsem, add=True)

    def wait_scatter(j, k):
      pltpu.make_async_copy(
          bufs[k], acc_sh.at[dst_v.at[pl.ds(j * ch, ch)]], ssems[k]).wait()

    for k in range(NBUF):
      gather(k, k)

    @pl.loop(0, n_chunks, step=NBUF)
    def _(j):
      for k in range(NBUF):
        wait_gather(j + k, k)
        scatter(j + k, k)
      for k in range(NBUF):
        @pl.when(j + k + NBUF < n_chunks)
        def _(k=k):
          wait_scatter(j + k, k)
          gather(j + k + NBUF, k)

    for k in range(NBUF):
      wait_scatter(n_chunks - NBUF + k, k)
    if with_deg:
      @pl.loop(0, n_chunks)
      def _(j):
        pltpu.make_async_copy(
            ones_v, deg_sh.at[dst_v.at[pl.ds(j * ch, ch)]], dsem).wait()

    plsc.subcore_barrier()
    pltpu.sync_copy(acc_sh.at[rsl], agg_out.at[c, rsl])
    if with_deg:
      pltpu.sync_copy(deg_sh.at[rsl], deg_out.at[c, rsl])

  return pl.kernel(
      body, out_type=out_type, mesh=mesh, scratch_types=scratch,
      compiler_params=pltpu.CompilerParams(use_tc_tiling_on_sc=False))


def _proj_call(x, wl, blk):
  n, k = x.shape
  d = wl.shape[1]

  def body(x_ref, wl_ref, p_ref):
    p_ref[...] = jnp.dot(x_ref[...], wl_ref[...],
                         preferred_element_type=jnp.float32)

  return pl.pallas_call(
      body,
      grid=(n // blk,),
      in_specs=[
          pl.BlockSpec((blk, k), lambda i: (i, 0)),
          pl.BlockSpec((k, d), lambda i: (0, 0)),
      ],
      out_specs=pl.BlockSpec((blk, d), lambda i: (i, 0)),
      out_shape=jax.ShapeDtypeStruct((n, d), jnp.float32),
  )(x, wl)


def _mid_layer_call(agg, degp, r, b, wl, blk, first):
  _, n, d = agg.shape
  do = wl.shape[1]
  nblk = n // blk
  aggf = agg.reshape(NC * n, d)
  gf = degp.reshape(NC * n, 16) if first else degp

  def body(a0_ref, a1_ref, g0_ref, g1_ref, r_ref, b_ref, wl_ref, *o_refs):
    a = a0_ref[...] + a1_ref[...]
    if first:
      deg = jnp.maximum(g0_ref[...] + g1_ref[...], 1.0)
      dinv = 1.0 / deg
    else:
      dinv = g0_ref[...]
    h = jnp.maximum(a * dinv[:, 0:1] + r_ref[...] + b_ref[...], 0.0)
    o_refs[0][...] = h
    o_refs[1][...] = jnp.dot(h, wl_ref[...], preferred_element_type=jnp.float32)
    if first:
      o_refs[2][...] = dinv

  g_spec0 = pl.BlockSpec((blk, 16), lambda i: (i, 0))
  g_spec1 = (pl.BlockSpec((blk, 16), lambda i: (i + nblk, 0)) if first
             else g_spec0)
  out_specs = [pl.BlockSpec((blk, d), lambda i: (i, 0)),
               pl.BlockSpec((blk, do), lambda i: (i, 0))]
  out_shape = [jax.ShapeDtypeStruct((n, d), jnp.float32),
               jax.ShapeDtypeStruct((n, do), jnp.float32)]
  if first:
    out_specs.append(pl.BlockSpec((blk, 16), lambda i: (i, 0)))
    out_shape.append(jax.ShapeDtypeStruct((n, 16), jnp.float32))

  return pl.pallas_call(
      body,
      grid=(nblk,),
      in_specs=[
          pl.BlockSpec((blk, d), lambda i: (i, 0)),
          pl.BlockSpec((blk, d), lambda i: (i + nblk, 0)),
          g_spec0,
          g_spec1,
          pl.BlockSpec((blk, d), lambda i: (i, 0)),
          pl.BlockSpec((1, d), lambda i: (0, 0)),
          pl.BlockSpec((d, do), lambda i: (0, 0)),
      ],
      out_specs=out_specs,
      out_shape=out_shape,
  )(aggf, aggf, gf, gf, r, b, wl)


def _final_call(agg, dinv, r, b, blk):
  _, n, d = agg.shape
  nblk = n // blk
  aggf = agg.reshape(NC * n, d)

  def body(a0_ref, a1_ref, g_ref, r_ref, b_ref, o_ref):
    a = a0_ref[...] + a1_ref[...]
    o_ref[...] = a * g_ref[...][:, 0:1] + r_ref[...] + b_ref[...]

  return pl.pallas_call(
      body,
      grid=(nblk,),
      in_specs=[
          pl.BlockSpec((blk, d), lambda i: (i, 0)),
          pl.BlockSpec((blk, d), lambda i: (i + nblk, 0)),
          pl.BlockSpec((blk, 16), lambda i: (i, 0)),
          pl.BlockSpec((blk, d), lambda i: (i, 0)),
          pl.BlockSpec((1, d), lambda i: (0, 0)),
      ],
      out_specs=pl.BlockSpec((blk, d), lambda i: (i, 0)),
      out_shape=jax.ShapeDtypeStruct((n, d), jnp.float32),
  )(aggf, aggf, dinv, r, b)


def kernel(x, edge_index, Wl1, Wr1, b1, Wl2, Wr2, b2, Wl3, Wr3, b3):
  n, d_in = x.shape
  e = edge_index.shape[1]
  d_h = Wl1.shape[1]
  n_cls = Wl3.shape[1]
  do = 16

  n_pad = -(-(n + 16) // 128) * 128
  e_tile = e // NW
  ch = CH
  n_chunks = -(-e_tile // (ch * NBUF)) * NBUF

  srcC = edge_index[0]
  dstC = edge_index[1]

  zeros_d = jnp.zeros((n_pad, d_h), jnp.float32)
  zeros_16 = jnp.zeros((n_pad, 16), jnp.float32)
  ones_16 = jnp.ones((ch, 16), jnp.float32)

  wl3p = jnp.zeros((d_h, do), jnp.float32).at[:, :n_cls].set(Wl3)
  wr3p = jnp.zeros((d_h, do), jnp.float32).at[:, :n_cls].set(Wr3)
  b3p = jnp.zeros((1, do), jnp.float32).at[0, :n_cls].set(b3)

  blk = n_pad // 8
  xp = jnp.pad(x, ((0, n_pad - n), (0, 0)))

  p1 = _proj_call(xp, Wl1, blk)
  agg1, degp = _make_sc_agg(n_pad, d_h, n_chunks, ch, e_tile, True)(
      p1, srcC, dstC, zeros_d, zeros_16, ones_16)
  r1 = _proj_call(xp, Wr1, blk)
  h1, p2, dinv = _mid_layer_call(
      agg1, degp, r1, b1.reshape(1, d_h), Wl2, blk, True)

  agg2, = _make_sc_agg(n_pad, d_h, n_chunks, ch, e_tile, False)(p2, srcC, dstC, zeros_d)
  r2 = _proj_call(h1, Wr2, blk)
  h2, p3 = _mid_layer_call(
      agg2, dinv, r2, b2.reshape(1, d_h), wl3p, blk, False)

  zeros_do = zeros_16 if do == 16 else jnp.zeros((n_pad, do), jnp.float32)
  agg3, = _make_sc_agg(n_pad, do, n_chunks, ch, e_tile, False)(p3, srcC, dstC, zeros_do)
  r3 = _proj_call(h2, wr3p, blk)
  out = _final_call(agg3, dinv, r3, b3p, blk)
  return out[:n, :n_cls]
